# Initial kernel scaffold; baseline (speedup 1.0000x reference)
import jax
import jax.numpy as jnp
from jax.experimental import pallas as pl


def kernel(a_hist, s_hist, s_delta, Rr, Rs, pe_W1, pe_b1, pe_W2, pe_b2, re_W1, re_b1, re_W2, re_b2, re_W3, re_b3, pp_W, pp_b, rp_W, rp_b, pr_W1, pr_b1, pr_W2, pr_b2):
    raise NotImplementedError("write your pallas kernel here")


# speedup vs baseline: 1.5940x; 1.5940x over previous
import jax
import jax.numpy as jnp
from jax.experimental import pallas as pl
from jax.experimental.pallas import tpu as pltpu
from functools import partial

NF_ = 64
H_ = 3
B_, N_, R_ = 2, 1024, 16384
PSTEP_ = 3
TR_ = 512
RT_ = R_ // TR_


def _relu(x):
    return jnp.maximum(x, 0.0)


def _dot(x, w):
    return jnp.dot(x, w, preferred_element_type=jnp.float32)


def _body(Rr_ref, Rs_ref, pe_in_ref, acs_ref,
          peW1_ref, peb1_ref, peW2_ref, peb2_ref,
          reA_ref, reC_ref, reb1_ref, reW2_ref, reb2_ref, reW3_ref, reb3_ref,
          rpWa_ref, rpb_ref, rpWb_ref, rpWc_ref,
          ppWa_ref, ppb_ref, ppWb_ref,
          prW1_ref, prb1_ref, prW2_ref, prb2_ref,
          out_ref,
          pf_scr, peproj_scr, agg_scr, rc_scr):
    p = pl.program_id(0)
    rt = pl.program_id(1)

    @pl.when((p == 0) & (rt == 0))
    def _():
        for b in range(B_):
            x = pe_in_ref[b]
            h = _relu(_dot(x, peW1_ref[...]) + peb1_ref[...])
            enc = _relu(_dot(h, peW2_ref[...]) + peb2_ref[...])
            pf_scr[b] = enc
            peproj_scr[b] = _dot(enc, ppWa_ref[...]) + ppb_ref[...]

    @pl.when(p == 0)
    def _():
        for b in range(B_):
            hr = _dot(Rr_ref[b], acs_ref[b])
            hs = _dot(Rs_ref[b], acs_ref[b])
            h1 = _relu(_dot(hr, reA_ref[...]) + _dot(hs, reC_ref[...])
                       + reb1_ref[...])
            h2 = _relu(_dot(h1, reW2_ref[...]) + reb2_ref[...])
            h3 = _relu(_dot(h2, reW3_ref[...]) + reb3_ref[...])
            rc_scr[b, pl.ds(rt * TR_, TR_), :] = (
                _dot(h3, rpWa_ref[...]) + rpb_ref[...])

    @pl.when(rt == 0)
    def _():
        agg_scr[...] = jnp.zeros_like(agg_scr)

    for b in range(B_):
        pf = pf_scr[b]
        er = _dot(Rr_ref[b], pf)
        es = _dot(Rs_ref[b], pf)
        rel = _relu(rc_scr[b, pl.ds(rt * TR_, TR_), :]
                    + _dot(er, rpWb_ref[...]) + _dot(es, rpWc_ref[...]))
        agg_scr[b] += jax.lax.dot_general(
            Rr_ref[b], rel,
            dimension_numbers=(((0,), (0,)), ((), ())),
            preferred_element_type=jnp.float32)

    @pl.when(rt == RT_ - 1)
    def _():
        for b in range(B_):
            pf_scr[b] = _relu(peproj_scr[b]
                              + _dot(agg_scr[b], ppWb_ref[...])
                              + pf_scr[b])

    @pl.when((p == PSTEP_ - 1) & (rt == RT_ - 1))
    def _():
        for b in range(B_):
            h = _relu(_dot(pf_scr[b], prW1_ref[...]) + prb1_ref[...])
            out_ref[b] = (_dot(h, prW2_ref[...]) + prb2_ref[...]
                          + acs_ref[b, :, 1:4])


@jax.jit
def kernel(a_hist, s_hist, s_delta, Rr, Rs,
           pe_W1, pe_b1, pe_W2, pe_b2,
           re_W1, re_b1, re_W2, re_b2, re_W3, re_b3,
           pp_W, pp_b, rp_W, rp_b,
           pr_W1, pr_b1, pr_W2, pr_b2):
    a = jnp.transpose(a_hist, (0, 2, 1))
    s = jnp.transpose(s_hist, (0, 2, 1, 3))
    sd = jnp.transpose(s_delta, (0, 2, 1, 3))
    sd_flat = sd.reshape(B_, N_, 3 * H_)
    pe_in = jnp.concatenate([sd_flat, a], axis=2)
    a_cur = a[:, :, -1]
    s_cur = s[:, :, -1, :]
    acs = jnp.concatenate([a_cur[..., None], s_cur], axis=2)

    reA = jnp.concatenate([re_W1[0:1], re_W1[2:5]], axis=0)
    reC = jnp.concatenate([re_W1[1:2], -re_W1[2:5]], axis=0)
    rpWa, rpWb, rpWc = rp_W[:NF_], rp_W[NF_:2 * NF_], rp_W[2 * NF_:]
    ppWa, ppWb = pp_W[:NF_], pp_W[NF_:]

    def row(v):
        return v.reshape(1, -1)

    full = lambda shape: pl.BlockSpec(shape, lambda p, rt: (0,) * len(shape))
    grid = (PSTEP_, RT_)

    in_specs = [
        pl.BlockSpec((B_, TR_, N_), lambda p, rt: (0, rt, 0)),
        pl.BlockSpec((B_, TR_, N_), lambda p, rt: (0, rt, 0)),
        full((B_, N_, 4 * H_)),
        full((B_, N_, 4)),
        full(pe_W1.shape), full((1, NF_)), full(pe_W2.shape), full((1, NF_)),
        full(reA.shape), full(reC.shape), full((1, NF_)),
        full(re_W2.shape), full((1, NF_)), full(re_W3.shape), full((1, NF_)),
        full(rpWa.shape), full((1, NF_)), full(rpWb.shape), full(rpWc.shape),
        full(ppWa.shape), full((1, NF_)), full(ppWb.shape),
        full(pr_W1.shape), full((1, NF_)), full(pr_W2.shape), full((1, 3)),
    ]

    pred = pl.pallas_call(
        _body,
        grid=grid,
        in_specs=in_specs,
        out_specs=full((B_, N_, 3)),
        out_shape=jax.ShapeDtypeStruct((B_, N_, 3), jnp.float32),
        scratch_shapes=[
            pltpu.VMEM((B_, N_, NF_), jnp.float32),
            pltpu.VMEM((B_, N_, NF_), jnp.float32),
            pltpu.VMEM((B_, N_, NF_), jnp.float32),
            pltpu.VMEM((B_, R_, NF_), jnp.float32),
        ],
        compiler_params=pltpu.CompilerParams(
            dimension_semantics=("arbitrary", "arbitrary"),
        ),
    )(Rr, Rs, pe_in, acs,
      pe_W1, row(pe_b1), pe_W2, row(pe_b2),
      reA, reC, row(re_b1), re_W2, row(re_b2), re_W3, row(re_b3),
      rpWa, row(rp_b), rpWb, rpWc,
      ppWa, row(pp_b), ppWb,
      pr_W1, row(pr_b1), pr_W2, row(pr_b2))
    return pred
